# Initial kernel scaffold; baseline (speedup 1.0000x reference)
import jax
import jax.numpy as jnp
from jax.experimental import pallas as pl


def kernel(news_x, company_x, edge_attr, mentions_src, mentions_dst, nn_edge_index, cc_edge_index, Wn, bn, Wc, bc, Wf, bf, gf, betaf, W1n_l, b1n, W1n_r, W1c_l, b1c, W1c_r, W2n_l, b2n, W2n_r, W2c_l, b2c, W2c_r, g_news, beta_news, g_comp, beta_comp, Wk1, bk1, Wk2, bk2):
    raise NotImplementedError("write your pallas kernel here")


# speedup vs baseline: 6.6753x; 6.6753x over previous
import functools

import jax
import jax.numpy as jnp
from jax import lax
from jax.experimental import pallas as pl
from jax.experimental.pallas import tpu as pltpu
from jax.experimental.pallas import tpu_sc as plsc

N_NEWS = 50000
N_COMP = 10000
H = 64
E_MEN = 800000
E_CC = 320000

NC = 2
NS = 16
NW = NC * NS
CHUNK = 128

MEN_CHUNKS = 196
MEN_BLK = 49
CC_CHUNKS = 80
CC_BLK = 40

ACC_ROWS = 10240
ZSTRIPE = ACC_ROWS // NS
WSTRIPE = 1000
NWT = N_COMP // WSTRIPE

_f32 = jnp.float32


def _mesh():
    return plsc.VectorSubcoreMesh(core_axis_name="c", subcore_axis_name="s",
                                  num_cores=NC, num_subcores=NS)


_SC_PARAMS = pltpu.CompilerParams(use_tc_tiling_on_sc=False)


def _zero_rows(rows_v, n):
    def zr(i, c):
        rows_v[i // 4, pl.ds((i % 4) * 16, 16)] = jnp.zeros((16,), _f32)
        return c
    lax.fori_loop(0, n * 4, zr, 0)


def _fill16(ref, n, value):
    def fb(i, c):
        ref[i, :] = jnp.full((16,), value, _f32)
        return c
    lax.fori_loop(0, n, fb, 0)


def _zero_shared(sid, zero64_v, zero16_v, acc_sh, cnt_shs):
    base = sid * ZSTRIPE
    def za(k, c):
        pltpu.sync_copy(zero64_v, acc_sh.at[pl.ds(base + k * CHUNK, CHUNK), :])
        return c
    lax.fori_loop(0, ZSTRIPE // CHUNK, za, 0)
    for cnt_sh in cnt_shs:
        def zc(k, c):
            pltpu.sync_copy(zero16_v, cnt_sh.at[pl.ds(base + k * CHUNK, CHUNK), :])
            return c
        lax.fori_loop(0, ZSTRIPE // CHUNK, zc, 0)


def _sc_mentions(news_h, msrc, mdst, ccdst):
    @functools.partial(
        pl.kernel,
        out_type=[
            jax.ShapeDtypeStruct((NC, N_COMP, H), _f32),
            jax.ShapeDtypeStruct((NC, N_COMP, 16), _f32),
            jax.ShapeDtypeStruct((NC, N_COMP, 16), _f32),
        ],
        mesh=_mesh(),
        compiler_params=_SC_PARAMS,
        scratch_types=[
            pltpu.VMEM((MEN_BLK, CHUNK), jnp.int32),
            pltpu.VMEM((MEN_BLK, CHUNK), jnp.int32),
            pltpu.VMEM((CC_BLK, CHUNK), jnp.int32),
            pltpu.VMEM((CHUNK, H), _f32),
            pltpu.VMEM((CHUNK, 16), _f32),
            pltpu.VMEM_SHARED((ACC_ROWS, H), _f32),
            pltpu.VMEM_SHARED((ACC_ROWS, 16), _f32),
            pltpu.VMEM_SHARED((ACC_ROWS, 16), _f32),
            pltpu.SemaphoreType.DMA,
        ],
    )
    def k(news_hbm, msrc_hbm, mdst_hbm, ccdst_hbm,
          msum_hbm, mcnt_hbm, ccnt_hbm,
          src_v, dst_v, ccdst_v, rows_v, ones_v,
          acc_sh, mcnt_sh, ccnt_sh, sem):
        cid = lax.axis_index("c")
        sid = lax.axis_index("s")
        wid = sid * NC + cid

        _zero_rows(rows_v, CHUNK)
        _fill16(ones_v, CHUNK, 0.0)
        _zero_shared(sid, rows_v, ones_v, acc_sh, (mcnt_sh, ccnt_sh))
        _fill16(ones_v, CHUNK, 1.0)
        plsc.subcore_barrier()

        def ccblk(b, c):
            pltpu.sync_copy(
                ccdst_hbm.at[pl.ds(wid * CC_CHUNKS + b * CC_BLK, CC_BLK), :],
                ccdst_v)
            def ccb(j, c2):
                pltpu.sync_copy(ones_v, ccnt_sh.at[ccdst_v.at[j]], add=True)
                return c2
            return lax.fori_loop(0, CC_BLK, ccb, c)
        lax.fori_loop(0, CC_CHUNKS // CC_BLK, ccblk, 0)

        def mblk(b, c):
            base = wid * MEN_CHUNKS + b * MEN_BLK
            pltpu.sync_copy(msrc_hbm.at[pl.ds(base, MEN_BLK), :], src_v)
            pltpu.sync_copy(mdst_hbm.at[pl.ds(base, MEN_BLK), :], dst_v)
            def mb(j, c2):
                pltpu.async_copy(news_hbm.at[src_v.at[j]], rows_v, sem).wait()
                pltpu.sync_copy(rows_v, acc_sh.at[dst_v.at[j]], add=True)
                pltpu.sync_copy(ones_v, mcnt_sh.at[dst_v.at[j]], add=True)
                return c2
            return lax.fori_loop(0, MEN_BLK, mb, c)
        lax.fori_loop(0, MEN_CHUNKS // MEN_BLK, mblk, 0)

        plsc.subcore_barrier()

        @pl.when(sid < NWT)
        def _():
            wbase = sid * WSTRIPE
            pltpu.sync_copy(acc_sh.at[pl.ds(wbase, WSTRIPE), :],
                            msum_hbm.at[cid, pl.ds(wbase, WSTRIPE), :])
            pltpu.sync_copy(mcnt_sh.at[pl.ds(wbase, WSTRIPE), :],
                            mcnt_hbm.at[cid, pl.ds(wbase, WSTRIPE), :])
            pltpu.sync_copy(ccnt_sh.at[pl.ds(wbase, WSTRIPE), :],
                            ccnt_hbm.at[cid, pl.ds(wbase, WSTRIPE), :])

    return k(news_h, msrc, mdst, ccdst)


def _sc_ccsum(table, ccsrc, ccdst):
    @functools.partial(
        pl.kernel,
        out_type=jax.ShapeDtypeStruct((NC, N_COMP, H), _f32),
        mesh=_mesh(),
        compiler_params=_SC_PARAMS,
        scratch_types=[
            pltpu.VMEM((CC_BLK, CHUNK), jnp.int32),
            pltpu.VMEM((CC_BLK, CHUNK), jnp.int32),
            pltpu.VMEM((CHUNK, H), _f32),
            pltpu.VMEM_SHARED((ACC_ROWS, H), _f32),
            pltpu.SemaphoreType.DMA,
        ],
    )
    def k(table_hbm, src_hbm, dst_hbm, out_hbm, src_v, dst_v, rows_v, acc_sh,
          sem):
        cid = lax.axis_index("c")
        sid = lax.axis_index("s")
        wid = sid * NC + cid

        _zero_rows(rows_v, CHUNK)
        zbase = sid * ZSTRIPE
        def za(k_, c):
            pltpu.sync_copy(rows_v, acc_sh.at[pl.ds(zbase + k_ * CHUNK, CHUNK), :])
            return c
        lax.fori_loop(0, ZSTRIPE // CHUNK, za, 0)
        plsc.subcore_barrier()

        def cblk(b, c):
            base = wid * CC_CHUNKS + b * CC_BLK
            pltpu.sync_copy(src_hbm.at[pl.ds(base, CC_BLK), :], src_v)
            pltpu.sync_copy(dst_hbm.at[pl.ds(base, CC_BLK), :], dst_v)
            def cb(j, c2):
                pltpu.async_copy(table_hbm.at[src_v.at[j]], rows_v, sem).wait()
                pltpu.sync_copy(rows_v, acc_sh.at[dst_v.at[j]], add=True)
                return c2
            return lax.fori_loop(0, CC_BLK, cb, c)
        lax.fori_loop(0, CC_CHUNKS // CC_BLK, cblk, 0)

        plsc.subcore_barrier()

        @pl.when(sid < NWT)
        def _():
            wbase = sid * WSTRIPE
            pltpu.sync_copy(acc_sh.at[pl.ds(wbase, WSTRIPE), :],
                            out_hbm.at[cid, pl.ds(wbase, WSTRIPE), :])

    return k(table, ccsrc, ccdst)


def _ln(x, g, b):
    m = jnp.mean(x, axis=-1, keepdims=True)
    xc = x - m
    v = jnp.mean(xc * xc, axis=-1, keepdims=True)
    return xc / jnp.sqrt(v + 1e-5) * g + b


def _tc_news_proj(news_x, Wn, bn2):
    BLK = 5000
    def body(x_ref, w_ref, b_ref, o_ref):
        o_ref[...] = jnp.maximum(
            jnp.dot(x_ref[...], w_ref[...], preferred_element_type=_f32)
            + b_ref[...], 0.0)
    return pl.pallas_call(
        body,
        grid=(N_NEWS // BLK,),
        in_specs=[
            pl.BlockSpec((BLK, 128), lambda i: (i, 0)),
            pl.BlockSpec((128, H), lambda i: (0, 0)),
            pl.BlockSpec((1, H), lambda i: (0, 0)),
        ],
        out_specs=pl.BlockSpec((BLK, H), lambda i: (i, 0)),
        out_shape=jax.ShapeDtypeStruct((N_NEWS, H), _f32),
    )(news_x, Wn, bn2)


def _tc_fuse(company_x, Wc, bc2, msum2, mcnt2, Wf, bf2, gf2, betaf2):
    def body(cx_ref, wc_ref, bc_ref, ms_ref, mc_ref, wf_ref, bf_ref,
             g_ref, b_ref, o_ref):
        ch = jnp.maximum(
            jnp.dot(cx_ref[...], wc_ref[...], preferred_element_type=_f32)
            + bc_ref[...], 0.0)
        msum = ms_ref[0] + ms_ref[1]
        cnt = mc_ref[0, :, 0:1] + mc_ref[1, :, 0:1]
        agg = msum / jnp.maximum(cnt, 1.0)
        z = (jnp.dot(ch, wf_ref[0:H, :], preferred_element_type=_f32)
             + jnp.dot(agg, wf_ref[H:2 * H, :], preferred_element_type=_f32)
             + bf_ref[...])
        o_ref[...] = _ln(jnp.maximum(z, 0.0), g_ref[...], b_ref[...])
    return pl.pallas_call(
        body,
        out_shape=jax.ShapeDtypeStruct((N_COMP, H), _f32),
    )(company_x, Wc, bc2, msum2, mcnt2, Wf, bf2, gf2, betaf2)


def _tc_conv(s2, cnt2, x, Wl, bl2, Wr, g2, b2):
    def body(s_ref, c_ref, x_ref, wl_ref, bl_ref, wr_ref, g_ref, b_ref, o_ref):
        ssum = s_ref[0] + s_ref[1]
        cnt = c_ref[0, :, 0:1] + c_ref[1, :, 0:1]
        agg = ssum / jnp.maximum(cnt, 1.0)
        z = (jnp.dot(agg, wl_ref[...], preferred_element_type=_f32)
             + bl_ref[...]
             + jnp.dot(x_ref[...], wr_ref[...], preferred_element_type=_f32))
        o_ref[...] = _ln(jnp.maximum(z, 0.0), g_ref[...], b_ref[...])
    return pl.pallas_call(
        body,
        out_shape=jax.ShapeDtypeStruct((N_COMP, H), _f32),
    )(s2, cnt2, x, Wl, bl2, Wr, g2, b2)


def _tc_out(s2, cnt2, x, Wl, bl2, Wr, g2, b2, Wk1, bk12, Wk2t, bk2s):
    def body(s_ref, c_ref, x_ref, wl_ref, bl_ref, wr_ref, g_ref, b_ref,
             wk1_ref, bk1_ref, wk2_ref, bk2_ref, o_ref):
        ssum = s_ref[0] + s_ref[1]
        cnt = c_ref[0, :, 0:1] + c_ref[1, :, 0:1]
        agg = ssum / jnp.maximum(cnt, 1.0)
        z = (jnp.dot(agg, wl_ref[...], preferred_element_type=_f32)
             + bl_ref[...]
             + jnp.dot(x_ref[...], wr_ref[...], preferred_element_type=_f32))
        co = _ln(jnp.maximum(z, 0.0), g_ref[...], b_ref[...])
        h = jnp.maximum(
            jnp.dot(co, wk1_ref[...], preferred_element_type=_f32)
            + bk1_ref[...], 0.0)
        o_ref[...] = (jnp.sum(h * wk2_ref[...], axis=1, keepdims=True)
                      + bk2_ref[...])
    return pl.pallas_call(
        body,
        out_shape=jax.ShapeDtypeStruct((N_COMP, 1), _f32),
    )(s2, cnt2, x, Wl, bl2, Wr, g2, b2, Wk1, bk12, Wk2t, bk2s)


def _pad_edges(idx, total, pad_value):
    n = total - idx.shape[0]
    return jnp.concatenate(
        [idx.astype(jnp.int32), jnp.full((n,), pad_value, jnp.int32)]
    ).reshape(total // CHUNK, CHUNK)


def kernel(news_x, company_x, edge_attr, mentions_src, mentions_dst,
           nn_edge_index, cc_edge_index,
           Wn, bn, Wc, bc, Wf, bf, gf, betaf,
           W1n_l, b1n, W1n_r, W1c_l, b1c, W1c_r,
           W2n_l, b2n, W2n_r, W2c_l, b2c, W2c_r,
           g_news, beta_news, g_comp, beta_comp,
           Wk1, bk1, Wk2, bk2):
    e_men_pad = NW * MEN_CHUNKS * CHUNK
    e_cc_pad = NW * CC_CHUNKS * CHUNK
    msrc = _pad_edges(mentions_src, e_men_pad, 0)
    mdst = _pad_edges(mentions_dst, e_men_pad, N_COMP)
    ccsrc = _pad_edges(cc_edge_index[0], e_cc_pad, 0)
    ccdst = _pad_edges(cc_edge_index[1], e_cc_pad, N_COMP)

    news_h = _tc_news_proj(news_x, Wn, bn.reshape(1, H))
    msum2, mcnt2, ccnt2 = _sc_mentions(news_h, msrc, mdst, ccdst)
    fused = _tc_fuse(company_x, Wc, bc.reshape(1, H), msum2, mcnt2,
                     Wf, bf.reshape(1, H), gf.reshape(1, H),
                     betaf.reshape(1, H))
    s1 = _sc_ccsum(fused, ccsrc, ccdst)
    comp1 = _tc_conv(s1, ccnt2, fused, W1c_l, b1c.reshape(1, H), W1c_r,
                     g_comp.reshape(1, H), beta_comp.reshape(1, H))
    s2 = _sc_ccsum(comp1, ccsrc, ccdst)
    logits2 = _tc_out(s2, ccnt2, comp1, W2c_l, b2c.reshape(1, H), W2c_r,
                      g_comp.reshape(1, H), beta_comp.reshape(1, H),
                      Wk1, bk1.reshape(1, 32), Wk2.reshape(1, 32),
                      bk2.reshape(1, 1))
    return logits2[:, 0]


# speedup vs baseline: 8.3673x; 1.2535x over previous
import functools

import jax
import jax.numpy as jnp
from jax import lax
from jax.experimental import pallas as pl
from jax.experimental.pallas import tpu as pltpu
from jax.experimental.pallas import tpu_sc as plsc

N_NEWS = 50000
N_COMP = 10000
H = 64
E_MEN = 800000
E_CC = 320000

NC = 2
NS = 16
NW = NC * NS
CHUNK = 128

MEN_CHUNKS = 196
MEN_BLK = 28
CC_CHUNKS = 80
CC_BLK = 40
NBUF = 4

ACC_ROWS = 10240
ZSTRIPE = ACC_ROWS // NS
WSTRIPE = 1000
NWT = N_COMP // WSTRIPE

_f32 = jnp.float32


def _mesh():
    return plsc.VectorSubcoreMesh(core_axis_name="c", subcore_axis_name="s",
                                  num_cores=NC, num_subcores=NS)


_SC_PARAMS = pltpu.CompilerParams(use_tc_tiling_on_sc=False)


def _zero_rows(rows_v, n):
    def zr(i, c):
        rows_v[i // 4, pl.ds((i % 4) * 16, 16)] = jnp.zeros((16,), _f32)
        return c
    lax.fori_loop(0, n * 4, zr, 0)


def _fill16(ref, n, value):
    def fb(i, c):
        ref[i, :] = jnp.full((16,), value, _f32)
        return c
    lax.fori_loop(0, n, fb, 0)


def _zero_shared(sid, zero64_v, zero16_v, acc_sh, cnt_shs):
    base = sid * ZSTRIPE
    def za(k, c):
        pltpu.sync_copy(zero64_v, acc_sh.at[pl.ds(base + k * CHUNK, CHUNK), :])
        return c
    lax.fori_loop(0, ZSTRIPE // CHUNK, za, 0)
    for cnt_sh in cnt_shs:
        def zc(k, c):
            pltpu.sync_copy(zero16_v, cnt_sh.at[pl.ds(base + k * CHUNK, CHUNK), :])
            return c
        lax.fori_loop(0, ZSTRIPE // CHUNK, zc, 0)


def _sc_mentions(news_h, msrc, mdst, ccdst):
    @functools.partial(
        pl.kernel,
        out_type=[
            jax.ShapeDtypeStruct((NC, N_COMP, H), _f32),
            jax.ShapeDtypeStruct((NC, N_COMP, 16), _f32),
            jax.ShapeDtypeStruct((NC, N_COMP, 16), _f32),
        ],
        mesh=_mesh(),
        compiler_params=_SC_PARAMS,
        scratch_types=[
            pltpu.VMEM((MEN_BLK, CHUNK), jnp.int32),
            pltpu.VMEM((MEN_BLK, CHUNK), jnp.int32),
            pltpu.VMEM((CC_CHUNKS, CHUNK), jnp.int32),
            [pltpu.VMEM((CHUNK, H), _f32)] * NBUF,
            pltpu.VMEM((CHUNK, 16), _f32),
            pltpu.VMEM_SHARED((ACC_ROWS, H), _f32),
            pltpu.VMEM_SHARED((ACC_ROWS, 16), _f32),
            pltpu.VMEM_SHARED((ACC_ROWS, 16), _f32),
            [pltpu.SemaphoreType.DMA] * NBUF,
            pltpu.SemaphoreType.DMA,
            pltpu.SemaphoreType.DMA,
        ],
    )
    def k(news_hbm, msrc_hbm, mdst_hbm, ccdst_hbm,
          msum_hbm, mcnt_hbm, ccnt_hbm,
          src_v, dst_v, ccdst_v, rows, ones_v,
          acc_sh, mcnt_sh, ccnt_sh, gsem, osem, csem):
        cid = lax.axis_index("c")
        sid = lax.axis_index("s")
        wid = sid * NC + cid

        _zero_rows(rows[0], CHUNK)
        _fill16(ones_v, CHUNK, 0.0)
        _zero_shared(sid, rows[0], ones_v, acc_sh, (mcnt_sh, ccnt_sh))
        _fill16(ones_v, CHUNK, 1.0)
        plsc.subcore_barrier()

        pltpu.sync_copy(
            ccdst_hbm.at[pl.ds(wid * CC_CHUNKS, CC_CHUNKS), :], ccdst_v)
        def ccb(j, c2):
            pltpu.async_copy(ones_v, ccnt_sh.at[ccdst_v.at[j]], csem,
                             add=True)
            return c2
        lax.fori_loop(0, CC_CHUNKS, ccb, 0)

        def mblk(b, c):
            base = wid * MEN_CHUNKS + b * MEN_BLK
            pltpu.sync_copy(msrc_hbm.at[pl.ds(base, MEN_BLK), :], src_v)
            pltpu.sync_copy(mdst_hbm.at[pl.ds(base, MEN_BLK), :], dst_v)
            for u in range(NBUF):
                pltpu.async_copy(news_hbm.at[src_v.at[u]], rows[u], gsem[u])
            def mb(jj, c2):
                for u in range(NBUF):
                    j = jj * NBUF + u
                    pltpu.make_async_copy(
                        news_hbm.at[src_v.at[j]], rows[u], gsem[u]).wait()
                    pltpu.sync_copy(rows[u], acc_sh.at[dst_v.at[j]], add=True)
                    pltpu.async_copy(ones_v, mcnt_sh.at[dst_v.at[j]], osem,
                                     add=True)
                    nj = j + NBUF
                    @pl.when(nj < MEN_BLK)
                    def _():
                        pltpu.async_copy(news_hbm.at[src_v.at[nj]], rows[u],
                                         gsem[u])
                return c2
            lax.fori_loop(0, MEN_BLK // NBUF, mb, 0)
            def dr(i, c2):
                pltpu.make_async_copy(ones_v, mcnt_sh.at[dst_v.at[0]],
                                      osem).wait()
                return c2
            return lax.fori_loop(0, MEN_BLK, dr, c)
        lax.fori_loop(0, MEN_CHUNKS // MEN_BLK, mblk, 0)

        def drcc(i, c2):
            pltpu.make_async_copy(ones_v, ccnt_sh.at[ccdst_v.at[0]],
                                  csem).wait()
            return c2
        lax.fori_loop(0, CC_CHUNKS, drcc, 0)

        plsc.subcore_barrier()

        @pl.when(sid < NWT)
        def _():
            wbase = sid * WSTRIPE
            pltpu.sync_copy(acc_sh.at[pl.ds(wbase, WSTRIPE), :],
                            msum_hbm.at[cid, pl.ds(wbase, WSTRIPE), :])
            pltpu.sync_copy(mcnt_sh.at[pl.ds(wbase, WSTRIPE), :],
                            mcnt_hbm.at[cid, pl.ds(wbase, WSTRIPE), :])
            pltpu.sync_copy(ccnt_sh.at[pl.ds(wbase, WSTRIPE), :],
                            ccnt_hbm.at[cid, pl.ds(wbase, WSTRIPE), :])

    return k(news_h, msrc, mdst, ccdst)


def _sc_ccsum(table, ccsrc, ccdst):
    @functools.partial(
        pl.kernel,
        out_type=jax.ShapeDtypeStruct((NC, N_COMP, H), _f32),
        mesh=_mesh(),
        compiler_params=_SC_PARAMS,
        scratch_types=[
            pltpu.VMEM((CC_BLK, CHUNK), jnp.int32),
            pltpu.VMEM((CC_BLK, CHUNK), jnp.int32),
            [pltpu.VMEM((CHUNK, H), _f32)] * NBUF,
            pltpu.VMEM_SHARED((ACC_ROWS, H), _f32),
            [pltpu.SemaphoreType.DMA] * NBUF,
        ],
    )
    def k(table_hbm, src_hbm, dst_hbm, out_hbm, src_v, dst_v, rows, acc_sh,
          gsem):
        cid = lax.axis_index("c")
        sid = lax.axis_index("s")
        wid = sid * NC + cid

        _zero_rows(rows[0], CHUNK)
        zbase = sid * ZSTRIPE
        def za(k_, c):
            pltpu.sync_copy(rows[0], acc_sh.at[pl.ds(zbase + k_ * CHUNK, CHUNK), :])
            return c
        lax.fori_loop(0, ZSTRIPE // CHUNK, za, 0)
        plsc.subcore_barrier()

        def cblk(b, c):
            base = wid * CC_CHUNKS + b * CC_BLK
            pltpu.sync_copy(src_hbm.at[pl.ds(base, CC_BLK), :], src_v)
            pltpu.sync_copy(dst_hbm.at[pl.ds(base, CC_BLK), :], dst_v)
            for u in range(NBUF):
                pltpu.async_copy(table_hbm.at[src_v.at[u]], rows[u], gsem[u])
            def cb(jj, c2):
                for u in range(NBUF):
                    j = jj * NBUF + u
                    pltpu.make_async_copy(
                        table_hbm.at[src_v.at[j]], rows[u], gsem[u]).wait()
                    pltpu.sync_copy(rows[u], acc_sh.at[dst_v.at[j]], add=True)
                    nj = j + NBUF
                    @pl.when(nj < CC_BLK)
                    def _():
                        pltpu.async_copy(table_hbm.at[src_v.at[nj]], rows[u],
                                         gsem[u])
                return c2
            return lax.fori_loop(0, CC_BLK // NBUF, cb, c)
        lax.fori_loop(0, CC_CHUNKS // CC_BLK, cblk, 0)

        plsc.subcore_barrier()

        @pl.when(sid < NWT)
        def _():
            wbase = sid * WSTRIPE
            pltpu.sync_copy(acc_sh.at[pl.ds(wbase, WSTRIPE), :],
                            out_hbm.at[cid, pl.ds(wbase, WSTRIPE), :])

    return k(table, ccsrc, ccdst)


def _ln(x, g, b):
    m = jnp.mean(x, axis=-1, keepdims=True)
    xc = x - m
    v = jnp.mean(xc * xc, axis=-1, keepdims=True)
    return xc / jnp.sqrt(v + 1e-5) * g + b


def _tc_news_proj(news_x, Wn, bn2):
    BLK = 5000
    def body(x_ref, w_ref, b_ref, o_ref):
        o_ref[...] = jnp.maximum(
            jnp.dot(x_ref[...], w_ref[...], preferred_element_type=_f32)
            + b_ref[...], 0.0)
    return pl.pallas_call(
        body,
        grid=(N_NEWS // BLK,),
        in_specs=[
            pl.BlockSpec((BLK, 128), lambda i: (i, 0)),
            pl.BlockSpec((128, H), lambda i: (0, 0)),
            pl.BlockSpec((1, H), lambda i: (0, 0)),
        ],
        out_specs=pl.BlockSpec((BLK, H), lambda i: (i, 0)),
        out_shape=jax.ShapeDtypeStruct((N_NEWS, H), _f32),
    )(news_x, Wn, bn2)


def _tc_fuse(company_x, Wc, bc2, msum2, mcnt2, Wf, bf2, gf2, betaf2):
    def body(cx_ref, wc_ref, bc_ref, ms_ref, mc_ref, wf_ref, bf_ref,
             g_ref, b_ref, o_ref):
        ch = jnp.maximum(
            jnp.dot(cx_ref[...], wc_ref[...], preferred_element_type=_f32)
            + bc_ref[...], 0.0)
        msum = ms_ref[0] + ms_ref[1]
        cnt = mc_ref[0, :, 0:1] + mc_ref[1, :, 0:1]
        agg = msum / jnp.maximum(cnt, 1.0)
        z = (jnp.dot(ch, wf_ref[0:H, :], preferred_element_type=_f32)
             + jnp.dot(agg, wf_ref[H:2 * H, :], preferred_element_type=_f32)
             + bf_ref[...])
        o_ref[...] = _ln(jnp.maximum(z, 0.0), g_ref[...], b_ref[...])
    return pl.pallas_call(
        body,
        out_shape=jax.ShapeDtypeStruct((N_COMP, H), _f32),
    )(company_x, Wc, bc2, msum2, mcnt2, Wf, bf2, gf2, betaf2)


def _tc_conv(s2, cnt2, x, Wl, bl2, Wr, g2, b2):
    def body(s_ref, c_ref, x_ref, wl_ref, bl_ref, wr_ref, g_ref, b_ref, o_ref):
        ssum = s_ref[0] + s_ref[1]
        cnt = c_ref[0, :, 0:1] + c_ref[1, :, 0:1]
        agg = ssum / jnp.maximum(cnt, 1.0)
        z = (jnp.dot(agg, wl_ref[...], preferred_element_type=_f32)
             + bl_ref[...]
             + jnp.dot(x_ref[...], wr_ref[...], preferred_element_type=_f32))
        o_ref[...] = _ln(jnp.maximum(z, 0.0), g_ref[...], b_ref[...])
    return pl.pallas_call(
        body,
        out_shape=jax.ShapeDtypeStruct((N_COMP, H), _f32),
    )(s2, cnt2, x, Wl, bl2, Wr, g2, b2)


def _tc_out(s2, cnt2, x, Wl, bl2, Wr, g2, b2, Wk1, bk12, Wk2t, bk2s):
    def body(s_ref, c_ref, x_ref, wl_ref, bl_ref, wr_ref, g_ref, b_ref,
             wk1_ref, bk1_ref, wk2_ref, bk2_ref, o_ref):
        ssum = s_ref[0] + s_ref[1]
        cnt = c_ref[0, :, 0:1] + c_ref[1, :, 0:1]
        agg = ssum / jnp.maximum(cnt, 1.0)
        z = (jnp.dot(agg, wl_ref[...], preferred_element_type=_f32)
             + bl_ref[...]
             + jnp.dot(x_ref[...], wr_ref[...], preferred_element_type=_f32))
        co = _ln(jnp.maximum(z, 0.0), g_ref[...], b_ref[...])
        h = jnp.maximum(
            jnp.dot(co, wk1_ref[...], preferred_element_type=_f32)
            + bk1_ref[...], 0.0)
        o_ref[...] = (jnp.sum(h * wk2_ref[...], axis=1, keepdims=True)
                      + bk2_ref[...])
    return pl.pallas_call(
        body,
        out_shape=jax.ShapeDtypeStruct((N_COMP, 1), _f32),
    )(s2, cnt2, x, Wl, bl2, Wr, g2, b2, Wk1, bk12, Wk2t, bk2s)


def _pad_edges(idx, total, pad_value):
    n = total - idx.shape[0]
    return jnp.concatenate(
        [idx.astype(jnp.int32), jnp.full((n,), pad_value, jnp.int32)]
    ).reshape(total // CHUNK, CHUNK)


def kernel(news_x, company_x, edge_attr, mentions_src, mentions_dst,
           nn_edge_index, cc_edge_index,
           Wn, bn, Wc, bc, Wf, bf, gf, betaf,
           W1n_l, b1n, W1n_r, W1c_l, b1c, W1c_r,
           W2n_l, b2n, W2n_r, W2c_l, b2c, W2c_r,
           g_news, beta_news, g_comp, beta_comp,
           Wk1, bk1, Wk2, bk2):
    e_men_pad = NW * MEN_CHUNKS * CHUNK
    e_cc_pad = NW * CC_CHUNKS * CHUNK
    msrc = _pad_edges(mentions_src, e_men_pad, 0)
    mdst = _pad_edges(mentions_dst, e_men_pad, N_COMP)
    ccsrc = _pad_edges(cc_edge_index[0], e_cc_pad, 0)
    ccdst = _pad_edges(cc_edge_index[1], e_cc_pad, N_COMP)

    news_h = _tc_news_proj(news_x, Wn, bn.reshape(1, H))
    msum2, mcnt2, ccnt2 = _sc_mentions(news_h, msrc, mdst, ccdst)
    fused = _tc_fuse(company_x, Wc, bc.reshape(1, H), msum2, mcnt2,
                     Wf, bf.reshape(1, H), gf.reshape(1, H),
                     betaf.reshape(1, H))
    s1 = _sc_ccsum(fused, ccsrc, ccdst)
    comp1 = _tc_conv(s1, ccnt2, fused, W1c_l, b1c.reshape(1, H), W1c_r,
                     g_comp.reshape(1, H), beta_comp.reshape(1, H))
    s2 = _sc_ccsum(comp1, ccsrc, ccdst)
    logits2 = _tc_out(s2, ccnt2, comp1, W2c_l, b2c.reshape(1, H), W2c_r,
                      g_comp.reshape(1, H), beta_comp.reshape(1, H),
                      Wk1, bk1.reshape(1, 32), Wk2.reshape(1, 32),
                      bk2.reshape(1, 1))
    return logits2[:, 0]
